# Initial kernel scaffold; baseline (speedup 1.0000x reference)
import jax
import jax.numpy as jnp
from jax.experimental import pallas as pl


def kernel(nodes, edges, supports, q, xyz):
    raise NotImplementedError("write your pallas kernel here")


# speedup vs baseline: 27.5097x; 27.5097x over previous
import functools

import jax
import jax.numpy as jnp
from jax import lax
from jax.experimental import pallas as pl
from jax.experimental.pallas import tpu as pltpu
from jax.experimental.pallas import tpu_sc as plsc


def _sc_call(u, v, q, x, y, z, sup_pad, *, N, E, NW, R, Npad, C):
    EPW = E // NW
    NCH = EPW // C
    NVR = R // 16
    mesh = plsc.VectorSubcoreMesh(core_axis_name="c", subcore_axis_name="s")
    NC = 2

    @functools.partial(
        pl.kernel,
        mesh=mesh,
        compiler_params=pltpu.CompilerParams(needs_layout_passes=False),
        out_type=(
            jax.ShapeDtypeStruct((4, NW, Npad), jnp.float32),
            jax.ShapeDtypeStruct((Npad,), jnp.int32),
        ),
        scratch_types=[
            pltpu.VMEM((N,), jnp.float32),
            pltpu.VMEM((Npad,), jnp.float32),
            pltpu.VMEM((C,), jnp.int32),
            pltpu.VMEM((C,), jnp.int32),
            pltpu.VMEM((C,), jnp.float32),
            pltpu.VMEM((R,), jnp.int32),
            pltpu.VMEM((R,), jnp.int32),
            pltpu.VMEM((R,), jnp.int32),
            pltpu.SemaphoreType.DMA,
        ],
    )
    def sc_kernel(u_hbm, v_hbm, q_hbm, x_hbm, y_hbm, z_hbm, sup_hbm,
                  partials_hbm, order_hbm,
                  coord, acc, ub, vb, qb, sbuf, posb, idb, sem):
        wid = lax.axis_index("s") * NC + lax.axis_index("c")

        def count_range(r, vsum):
            pltpu.sync_copy(sup_hbm.at[pl.ds(r * R, R)], sbuf)

            def acc16(i, vs):
                return vs + sbuf[pl.ds(i * 16, 16)]

            return lax.fori_loop(0, NVR, acc16, vsum)

        zero_vec = jnp.zeros((16,), jnp.int32)
        vsum_before = lax.fori_loop(0, wid, count_range, zero_vec)
        ones_before = jnp.sum(vsum_before)
        vsum_total = lax.fori_loop(wid, NW, count_range, vsum_before)
        total_ones = jnp.sum(vsum_total)
        num_free = Npad - total_ones
        free_before = wid * R - ones_before
        fixed_before = ones_before

        pltpu.sync_copy(sup_hbm.at[pl.ds(wid * R, R)], sbuf)

        def rank16(i, offs):
            free_off, fixed_off = offs
            s16 = sbuf[pl.ds(i * 16, 16)]
            m = s16 == 0
            mi = m.astype(jnp.int32)
            cfree = lax.cumsum(mi)
            cfix = lax.cumsum(1 - mi)
            pos = jnp.where(
                m,
                free_before + free_off + cfree - 1,
                num_free + fixed_before + fixed_off + cfix - 1,
            )
            ids = wid * R + i * 16 + lax.iota(jnp.int32, 16)
            posb[pl.ds(i * 16, 16)] = pos
            idb[pl.ds(i * 16, 16)] = ids
            cnt = jnp.sum(mi)
            return free_off + cnt, fixed_off + (16 - cnt)

        lax.fori_loop(0, NVR, rank16, (jnp.int32(0), jnp.int32(0)))
        pltpu.async_copy(idb, order_hbm.at[posb], sem).wait()

        coord_srcs = (x_hbm, y_hbm, z_hbm)
        for p in range(4):
            if p < 3:
                pltpu.sync_copy(coord_srcs[p], coord)

            def zero16(i, _):
                acc[pl.ds(i * 16, 16)] = jnp.zeros((16,), jnp.float32)
                return 0

            lax.fori_loop(0, Npad // 16, zero16, 0)

            def chunk(c, _, p=p):
                base = wid * EPW + c * C
                pltpu.sync_copy(u_hbm.at[pl.ds(base, C)], ub)
                pltpu.sync_copy(v_hbm.at[pl.ds(base, C)], vb)
                pltpu.sync_copy(q_hbm.at[pl.ds(base, C)], qb)

                def edge16(j, _, p=p):
                    off = j * 16
                    u16 = ub[pl.ds(off, 16)]
                    v16 = vb[pl.ds(off, 16)]
                    q16 = qb[pl.ds(off, 16)]
                    if p < 3:
                        cu = plsc.load_gather(coord, [u16])
                        cv = plsc.load_gather(coord, [v16])
                        f = q16 * (cv - cu)
                        plsc.addupdate_scatter(acc, [v16], f)
                        plsc.addupdate_scatter(acc, [u16], -f)
                    else:
                        plsc.addupdate_scatter(acc, [u16], q16)
                        plsc.addupdate_scatter(acc, [v16], q16)
                    return 0

                lax.fori_loop(0, C // 16, edge16, 0)
                return 0

            lax.fori_loop(0, NCH, chunk, 0)
            pltpu.sync_copy(acc, partials_hbm.at[p, wid])

    return sc_kernel(u, v, q, x, y, z, sup_pad)


def _combine_call(partials, sup2d, *, NW, Npad):
    GRID = 8
    BL = Npad // GRID

    def body(pref, sref, oref):
        s = jnp.sum(pref[...], axis=1)
        m = (sref[...] == 0).astype(jnp.float32)
        oref[...] = s * m

    return pl.pallas_call(
        body,
        grid=(GRID,),
        in_specs=[
            pl.BlockSpec((4, NW, BL), lambda i: (0, 0, i)),
            pl.BlockSpec((1, BL), lambda i: (0, i)),
        ],
        out_specs=pl.BlockSpec((4, BL), lambda i: (0, i)),
        out_shape=jax.ShapeDtypeStruct((4, Npad), jnp.float32),
    )(partials, sup2d)


def kernel(nodes, edges, supports, q, xyz):
    N = supports.shape[0]
    E = q.shape[0]
    NW = 32
    R = (-(-N // NW) + 15) // 16 * 16
    Npad = R * NW
    C = 2000

    u = edges[:, 0]
    v = edges[:, 1]
    x = xyz[:, 0]
    y = xyz[:, 1]
    z = xyz[:, 2]
    sup_pad = jnp.concatenate(
        [supports, jnp.ones((Npad - N,), supports.dtype)])

    partials, order_pad = _sc_call(
        u, v, q, x, y, z, sup_pad, N=N, E=E, NW=NW, R=R, Npad=Npad, C=C)
    out4 = _combine_call(partials, sup_pad.reshape(1, Npad), NW=NW, Npad=Npad)

    r_free = out4[:3, :N].T
    diag = out4[3, :N]
    order = order_pad[:N]
    return r_free, diag, order


# speedup vs baseline: 37.8401x; 1.3755x over previous
import functools

import jax
import jax.numpy as jnp
from jax import lax
from jax.experimental import pallas as pl
from jax.experimental.pallas import tpu as pltpu
from jax.experimental.pallas import tpu_sc as plsc


def _sc_call(u, v, q, x, y, z, sup_pad, *, N, E, NW, R, Npad, C):
    EPW = E // NW
    NCH = EPW // C
    NVR = R // 16
    UNR = 5
    mesh = plsc.VectorSubcoreMesh(core_axis_name="c", subcore_axis_name="s")
    NC = 2

    @functools.partial(
        pl.kernel,
        mesh=mesh,
        compiler_params=pltpu.CompilerParams(needs_layout_passes=False),
        out_type=(
            jax.ShapeDtypeStruct((4, NW, Npad), jnp.float32),
            jax.ShapeDtypeStruct((Npad,), jnp.int32),
        ),
        scratch_types=[
            pltpu.VMEM((N,), jnp.float32),
            pltpu.VMEM((Npad,), jnp.float32),
            pltpu.VMEM((C,), jnp.int32),
            pltpu.VMEM((C,), jnp.int32),
            pltpu.VMEM((C,), jnp.int32),
            pltpu.VMEM((C,), jnp.int32),
            pltpu.VMEM((C,), jnp.float32),
            pltpu.VMEM((C,), jnp.float32),
            pltpu.VMEM((R,), jnp.int32),
            pltpu.VMEM((R,), jnp.int32),
            pltpu.VMEM((R,), jnp.int32),
            pltpu.SemaphoreType.DMA,
            pltpu.SemaphoreType.DMA,
            pltpu.SemaphoreType.DMA,
        ],
    )
    def sc_kernel(u_hbm, v_hbm, q_hbm, x_hbm, y_hbm, z_hbm, sup_hbm,
                  partials_hbm, order_hbm,
                  coord, acc, ub0, ub1, vb0, vb1, qb0, qb1,
                  sbuf, posb, idb, sem, semb0, semb1):
        ub = (ub0, ub1)
        vb = (vb0, vb1)
        qb = (qb0, qb1)
        wid = lax.axis_index("s") * NC + lax.axis_index("c")

        def count_range(r, vsum):
            pltpu.sync_copy(sup_hbm.at[pl.ds(r * R, R)], sbuf)

            def acc16(i, vs):
                return vs + sbuf[pl.ds(i * 16, 16)]

            return lax.fori_loop(0, NVR, acc16, vsum)

        zero_vec = jnp.zeros((16,), jnp.int32)
        vsum_before = lax.fori_loop(0, wid, count_range, zero_vec)
        ones_before = jnp.sum(vsum_before)
        vsum_total = lax.fori_loop(wid, NW, count_range, vsum_before)
        total_ones = jnp.sum(vsum_total)
        num_free = Npad - total_ones
        free_before = wid * R - ones_before
        fixed_before = ones_before

        pltpu.sync_copy(sup_hbm.at[pl.ds(wid * R, R)], sbuf)

        def rank16(i, offs):
            free_off, fixed_off = offs
            s16 = sbuf[pl.ds(i * 16, 16)]
            m = s16 == 0
            mi = m.astype(jnp.int32)
            cfree = lax.cumsum(mi)
            cfix = lax.cumsum(1 - mi)
            pos = jnp.where(
                m,
                free_before + free_off + cfree - 1,
                num_free + fixed_before + fixed_off + cfix - 1,
            )
            ids = wid * R + i * 16 + lax.iota(jnp.int32, 16)
            posb[pl.ds(i * 16, 16)] = pos
            idb[pl.ds(i * 16, 16)] = ids
            cnt = jnp.sum(mi)
            return free_off + cnt, fixed_off + (16 - cnt)

        lax.fori_loop(0, NVR, rank16, (jnp.int32(0), jnp.int32(0)))
        pltpu.async_copy(idb, order_hbm.at[posb], sem).wait()

        coord_srcs = (x_hbm, y_hbm, z_hbm)
        for p in range(4):
            if p < 3:
                pltpu.sync_copy(coord_srcs[p], coord)

            def zero16(i, _):
                acc[pl.ds(i * 16, 16)] = jnp.zeros((16,), jnp.float32)
                return 0

            lax.fori_loop(0, Npad // 16, zero16, 0)

            sems = (semb0, semb1)

            def issue(c, b):
                base = wid * EPW + c * C
                pltpu.async_copy(u_hbm.at[pl.ds(base, C)], ub[b], sems[b])
                pltpu.async_copy(v_hbm.at[pl.ds(base, C)], vb[b], sems[b])
                pltpu.async_copy(q_hbm.at[pl.ds(base, C)], qb[b], sems[b])

            def drain(b):
                pltpu.make_async_copy(
                    u_hbm.at[pl.ds(0, C)], ub[b], sems[b]).wait()
                pltpu.make_async_copy(
                    v_hbm.at[pl.ds(0, C)], vb[b], sems[b]).wait()
                pltpu.make_async_copy(
                    q_hbm.at[pl.ds(0, C)], qb[b], sems[b]).wait()

            def process(b, p=p):
                def edge16(j, _, p=p, b=b):
                    for jj in range(UNR):
                        off = (j * UNR + jj) * 16
                        u16 = ub[b][pl.ds(off, 16)]
                        v16 = vb[b][pl.ds(off, 16)]
                        q16 = qb[b][pl.ds(off, 16)]
                        if p < 3:
                            cu = plsc.load_gather(coord, [u16])
                            cv = plsc.load_gather(coord, [v16])
                            f = q16 * (cv - cu)
                            plsc.addupdate_scatter(acc, [v16], f)
                            plsc.addupdate_scatter(acc, [u16], -f)
                        else:
                            plsc.addupdate_scatter(acc, [u16], q16)
                            plsc.addupdate_scatter(acc, [v16], q16)
                    return 0

                lax.fori_loop(0, C // (16 * UNR), edge16, 0)

            issue(jnp.int32(0), 0)
            issue(jnp.int32(1), 1)

            def pair(g, _, p=p):
                for b in range(2):
                    c = g * 2 + b
                    drain(b)
                    process(b)

                    @pl.when(c + 2 < NCH)
                    def _():
                        issue(c + 2, b)

                return 0

            lax.fori_loop(0, NCH // 2, pair, 0)
            drain(0)
            process(0)
            pltpu.sync_copy(acc, partials_hbm.at[p, wid])

    return sc_kernel(u, v, q, x, y, z, sup_pad)


def _combine_call(partials, sup2d, *, NW, Npad):
    GRID = 8
    BL = Npad // GRID

    def body(pref, sref, oref):
        s = jnp.sum(pref[...], axis=1)
        m = (sref[...] == 0).astype(jnp.float32)
        oref[...] = s * m

    return pl.pallas_call(
        body,
        grid=(GRID,),
        in_specs=[
            pl.BlockSpec((4, NW, BL), lambda i: (0, 0, i)),
            pl.BlockSpec((1, BL), lambda i: (0, i)),
        ],
        out_specs=pl.BlockSpec((4, BL), lambda i: (0, i)),
        out_shape=jax.ShapeDtypeStruct((4, Npad), jnp.float32),
    )(partials, sup2d)


def kernel(nodes, edges, supports, q, xyz):
    N = supports.shape[0]
    E = q.shape[0]
    NW = 32
    R = (-(-N // NW) + 15) // 16 * 16
    Npad = R * NW
    C = 2000

    u = edges[:, 0]
    v = edges[:, 1]
    x = xyz[:, 0]
    y = xyz[:, 1]
    z = xyz[:, 2]
    sup_pad = jnp.concatenate(
        [supports, jnp.ones((Npad - N,), supports.dtype)])

    partials, order_pad = _sc_call(
        u, v, q, x, y, z, sup_pad, N=N, E=E, NW=NW, R=R, Npad=Npad, C=C)
    out4 = _combine_call(partials, sup_pad.reshape(1, Npad), NW=NW, Npad=Npad)

    r_free = out4[:3, :N].T
    diag = out4[3, :N]
    order = order_pad[:N]
    return r_free, diag, order


# speedup vs baseline: 48.6531x; 1.2858x over previous
import functools

import jax
import jax.numpy as jnp
from jax import lax
from jax.experimental import pallas as pl
from jax.experimental.pallas import tpu as pltpu
from jax.experimental.pallas import tpu_sc as plsc


def _sc_call(u, v, q, x, y, z, sup_pad, *, N, E, NW, R, Npad, C):
    EPW = E // NW
    NCH = EPW // C
    NVR = R // 16
    UNR = 5
    mesh = plsc.VectorSubcoreMesh(core_axis_name="c", subcore_axis_name="s")
    NC = 2

    @functools.partial(
        pl.kernel,
        mesh=mesh,
        compiler_params=pltpu.CompilerParams(needs_layout_passes=False),
        out_type=(
            jax.ShapeDtypeStruct((4, NW, Npad), jnp.float32),
            jax.ShapeDtypeStruct((Npad,), jnp.int32),
        ),
        scratch_types=[
            pltpu.VMEM((N,), jnp.float32),
            pltpu.VMEM((Npad,), jnp.float32),
            pltpu.VMEM((C,), jnp.int32),
            pltpu.VMEM((C,), jnp.int32),
            pltpu.VMEM((C,), jnp.int32),
            pltpu.VMEM((C,), jnp.int32),
            pltpu.VMEM((C,), jnp.float32),
            pltpu.VMEM((C,), jnp.float32),
            pltpu.VMEM((4 * R,), jnp.int32),
            pltpu.VMEM((R,), jnp.int32),
            pltpu.VMEM((R,), jnp.int32),
            pltpu.SemaphoreType.DMA,
            pltpu.SemaphoreType.DMA,
            pltpu.SemaphoreType.DMA,
        ],
    )
    def sc_kernel(u_hbm, v_hbm, q_hbm, x_hbm, y_hbm, z_hbm, sup_hbm,
                  partials_hbm, order_hbm,
                  coord, acc, ub0, ub1, vb0, vb1, qb0, qb1,
                  sbuf, posb, idb, sem, semb0, semb1):
        ub = (ub0, ub1)
        vb = (vb0, vb1)
        qb = (qb0, qb1)
        wid = lax.axis_index("s") * NC + lax.axis_index("c")

        zero_vec = jnp.zeros((16,), jnp.int32)
        bound = wid * NVR

        def count_group(g, carry):
            vpre, vtot = carry
            pltpu.sync_copy(sup_hbm.at[pl.ds(g * 4 * R, 4 * R)], sbuf)

            def acc16(i, c):
                vp, vt = c
                s16 = sbuf[pl.ds(i * 16, 16)]
                gv = g * (4 * NVR) + i
                return vp + jnp.where(gv < bound, s16, zero_vec), vt + s16

            return lax.fori_loop(0, 4 * NVR, acc16, (vpre, vtot))

        vsum_before, vsum_total = lax.fori_loop(
            0, NW // 4, count_group, (zero_vec, zero_vec))
        ones_before = jnp.sum(vsum_before)
        total_ones = jnp.sum(vsum_total)
        num_free = Npad - total_ones
        free_before = wid * R - ones_before
        fixed_before = ones_before

        pltpu.sync_copy(sup_hbm.at[pl.ds(wid * R, R)], sbuf.at[pl.ds(0, R)])

        def rank16(i, offs):
            free_off, fixed_off = offs
            s16 = sbuf[pl.ds(i * 16, 16)]
            m = s16 == 0
            mi = m.astype(jnp.int32)
            cfree = lax.cumsum(mi)
            cfix = lax.cumsum(1 - mi)
            pos = jnp.where(
                m,
                free_before + free_off + cfree - 1,
                num_free + fixed_before + fixed_off + cfix - 1,
            )
            ids = wid * R + i * 16 + lax.iota(jnp.int32, 16)
            posb[pl.ds(i * 16, 16)] = pos
            idb[pl.ds(i * 16, 16)] = ids
            cnt = jnp.sum(mi)
            return free_off + cnt, fixed_off + (16 - cnt)

        lax.fori_loop(0, NVR, rank16, (jnp.int32(0), jnp.int32(0)))
        pltpu.async_copy(idb, order_hbm.at[posb], sem).wait()

        coord_srcs = (x_hbm, y_hbm, z_hbm)
        for p in range(4):
            if p < 3:
                pltpu.sync_copy(coord_srcs[p], coord)

            @plsc.parallel_loop(0, Npad // 16, unroll=8)
            def zero16(i):
                acc[pl.ds(i * 16, 16)] = jnp.zeros((16,), jnp.float32)

            sems = (semb0, semb1)

            def issue(c, b):
                base = wid * EPW + c * C
                pltpu.async_copy(u_hbm.at[pl.ds(base, C)], ub[b], sems[b])
                pltpu.async_copy(v_hbm.at[pl.ds(base, C)], vb[b], sems[b])
                pltpu.async_copy(q_hbm.at[pl.ds(base, C)], qb[b], sems[b])

            def drain(b):
                pltpu.make_async_copy(
                    u_hbm.at[pl.ds(0, C)], ub[b], sems[b]).wait()
                pltpu.make_async_copy(
                    v_hbm.at[pl.ds(0, C)], vb[b], sems[b]).wait()
                pltpu.make_async_copy(
                    q_hbm.at[pl.ds(0, C)], qb[b], sems[b]).wait()

            def process(b, p=p):
                @plsc.parallel_loop(0, C // (16 * UNR), unroll=2)
                def edge16(j, p=p, b=b):
                    for jj in range(UNR):
                        off = (j * UNR + jj) * 16
                        u16 = ub[b][pl.ds(off, 16)]
                        v16 = vb[b][pl.ds(off, 16)]
                        q16 = qb[b][pl.ds(off, 16)]
                        if p < 3:
                            cu = plsc.load_gather(coord, [u16])
                            cv = plsc.load_gather(coord, [v16])
                            f = q16 * (cv - cu)
                            plsc.addupdate_scatter(acc, [v16], f)
                            plsc.addupdate_scatter(acc, [u16], -f)
                        else:
                            plsc.addupdate_scatter(acc, [u16], q16)
                            plsc.addupdate_scatter(acc, [v16], q16)

            issue(jnp.int32(0), 0)
            issue(jnp.int32(1), 1)

            def pair(g, _, p=p):
                for b in range(2):
                    c = g * 2 + b
                    drain(b)
                    process(b)

                    @pl.when(c + 2 < NCH)
                    def _():
                        issue(c + 2, b)

                return 0

            lax.fori_loop(0, NCH // 2, pair, 0)
            drain(0)
            process(0)
            pltpu.sync_copy(acc, partials_hbm.at[p, wid])

    return sc_kernel(u, v, q, x, y, z, sup_pad)


def _combine_call(partials, sup2d, *, NW, Npad):
    GRID = 8
    BL = Npad // GRID

    def body(pref, sref, oref):
        s = jnp.sum(pref[...], axis=1)
        m = (sref[...] == 0).astype(jnp.float32)
        oref[...] = s * m

    return pl.pallas_call(
        body,
        grid=(GRID,),
        in_specs=[
            pl.BlockSpec((4, NW, BL), lambda i: (0, 0, i)),
            pl.BlockSpec((1, BL), lambda i: (0, i)),
        ],
        out_specs=pl.BlockSpec((4, BL), lambda i: (0, i)),
        out_shape=jax.ShapeDtypeStruct((4, Npad), jnp.float32),
    )(partials, sup2d)


def kernel(nodes, edges, supports, q, xyz):
    N = supports.shape[0]
    E = q.shape[0]
    NW = 32
    R = (-(-N // NW) + 15) // 16 * 16
    Npad = R * NW
    C = 2000

    u = edges[:, 0]
    v = edges[:, 1]
    x = xyz[:, 0]
    y = xyz[:, 1]
    z = xyz[:, 2]
    sup_pad = jnp.concatenate(
        [supports, jnp.ones((Npad - N,), supports.dtype)])

    partials, order_pad = _sc_call(
        u, v, q, x, y, z, sup_pad, N=N, E=E, NW=NW, R=R, Npad=Npad, C=C)
    out4 = _combine_call(partials, sup_pad.reshape(1, Npad), NW=NW, Npad=Npad)

    r_free = out4[:3, :N].T
    diag = out4[3, :N]
    order = order_pad[:N]
    return r_free, diag, order
